# Initial kernel scaffold; baseline (speedup 1.0000x reference)
import jax
import jax.numpy as jnp
from jax.experimental import pallas as pl


def kernel(h_nodes, h_edges, W_edit, b_edit, W_stop, b_stop, edit_ij, edit_b, feas, stop_feas):
    raise NotImplementedError("write your pallas kernel here")


# speedup vs baseline: 3.1093x; 3.1093x over previous
import functools

import jax
import jax.numpy as jnp
from jax import lax
from jax.experimental import pallas as pl
from jax.experimental.pallas import tpu as pltpu
from jax.experimental.pallas import tpu_sc as plsc

_B, _N, _D, _E, _A = 32, 128, 256, 16, 4096
_L = 16
_NCHUNK = _A // _L
_ROWS_PER_DMA = 128
_DMA_GROUPS = 4
_DMAS_PER_GROUP = (_A // _ROWS_PER_DMA) // _DMA_GROUPS


def _tc_body(h_ref, w_ref, bias_ref, s_ref, stop_ref):
    h = h_ref[0]
    s = jnp.dot(h, w_ref[...], preferred_element_type=jnp.float32)
    s = s + bias_ref[...]
    s_ref[0] = s
    stop_ref[...] = jnp.full((1, 1, 128), jnp.sum(s[:, 8:9]), jnp.float32)


def _node_tables(h_nodes, w_cat, bias):
    return pl.pallas_call(
        _tc_body,
        grid=(_B,),
        in_specs=[
            pl.BlockSpec((1, _N, _D), lambda b: (b, 0, 0)),
            pl.BlockSpec((_D, 16), lambda b: (0, 0)),
            pl.BlockSpec((1, 16), lambda b: (0, 0)),
        ],
        out_specs=[
            pl.BlockSpec((1, _N, 16), lambda b: (b, 0, 0)),
            pl.BlockSpec((1, 1, 128), lambda b: (b, 0, 0)),
        ],
        out_shape=[
            jax.ShapeDtypeStruct((_B, _N, 16), jnp.float32),
            jax.ShapeDtypeStruct((_B, 1, 128), jnp.float32),
        ],
    )(h_nodes, w_cat, bias)


def _sc_edit_logits(he_flat, s_all, w_e, eidx, i_idx, j_idx, c_idx, feas):
    info = plsc.get_sparse_core_info()
    mesh = plsc.VectorSubcoreMesh(core_axis_name="c", subcore_axis_name="s")

    @functools.partial(
        pl.kernel,
        out_type=jax.ShapeDtypeStruct((_B, _A), jnp.float32),
        mesh=mesh,
        compiler_params=pltpu.CompilerParams(
            needs_layout_passes=False, use_tc_tiling_on_sc=False),
        scratch_types=[
            pltpu.VMEM((_A // _ROWS_PER_DMA, _ROWS_PER_DMA), jnp.int32),
            pltpu.VMEM((_A,), jnp.int32),
            pltpu.VMEM((_A,), jnp.int32),
            pltpu.VMEM((_A,), jnp.int32),
            pltpu.VMEM((_A,), jnp.int32),
            pltpu.VMEM((_N * 16,), jnp.float32),
            pltpu.VMEM((4 * _E,), jnp.float32),
            pltpu.VMEM((_A, _E), jnp.float32),
            pltpu.VMEM((_A,), jnp.float32),
            pltpu.SemaphoreType.DMA,
        ],
    )
    def k(he_hbm, s_hbm, we_hbm, eidx_hbm, i_hbm, j_hbm, c_hbm, feas_hbm,
          out_hbm, eidx_v, i_v, j_v, c_v, feas_v, s_v, we_v, rows_v, out_v,
          sem):
        wid = lax.axis_index("s") * info.num_cores + lax.axis_index("c")
        pltpu.sync_copy(eidx_hbm, eidx_v)
        pltpu.sync_copy(i_hbm, i_v)
        pltpu.sync_copy(j_hbm, j_v)
        pltpu.sync_copy(c_hbm, c_v)
        pltpu.sync_copy(feas_hbm.at[wid], feas_v)
        pltpu.sync_copy(s_hbm.at[wid], s_v)
        pltpu.sync_copy(we_hbm, we_v)

        he_b = he_hbm.at[wid]

        def gather_group(g, _):
            cps = []
            for q in range(_DMAS_PER_GROUP):
                r = g * _DMAS_PER_GROUP + q
                cps.append(pltpu.async_copy(
                    he_b.at[eidx_v.at[r]],
                    rows_v.at[pl.ds(r * _ROWS_PER_DMA, _ROWS_PER_DMA)],
                    sem))
            for cp in cps:
                cp.wait()
            return 0

        lax.fori_loop(0, _DMA_GROUPS, gather_group, 0)

        iota = lax.iota(jnp.int32, _L)
        minf = jnp.float32(-jnp.inf)

        def chunk(q, _):
            a0 = q * _L
            iv = i_v[pl.ds(a0, _L)]
            jv = j_v[pl.ds(a0, _L)]
            cv = c_v[pl.ds(a0, _L)]
            acc = plsc.load_gather(s_v, [iv * 16 + cv])
            acc = acc + plsc.load_gather(s_v, [jv * 16 + (cv + 4)])
            av = a0 + iota
            for e in range(_E):
                ev = jnp.full((_L,), e, jnp.int32)
                w = plsc.load_gather(we_v, [cv * _E + e])
                hv = plsc.load_gather(rows_v, [av, ev])
                acc = acc + hv * w
            fv = feas_v[pl.ds(a0, _L)]
            out_v[pl.ds(a0, _L)] = jnp.where(fv != 0, acc, minf)
            return 0

        lax.fori_loop(0, _NCHUNK, chunk, 0)
        pltpu.sync_copy(out_v, out_hbm.at[wid])

    return k(he_flat, s_all, w_e, eidx, i_idx, j_idx, c_idx, feas)


def kernel(h_nodes, h_edges, W_edit, b_edit, W_stop, b_stop, edit_ij, edit_b,
           feas, stop_feas):
    w_cat = jnp.concatenate(
        [W_edit[:_D], W_edit[_D:2 * _D], W_stop / _N,
         jnp.zeros((_D, 7), jnp.float32)], axis=1)
    bias = jnp.concatenate(
        [b_edit, jnp.zeros((4,), jnp.float32), b_stop / _N,
         jnp.zeros((7,), jnp.float32)])[None, :]
    w_e_t = W_edit[2 * _D:].T.reshape(4 * _E)

    i_idx = edit_ij[0]
    j_idx = edit_ij[1]
    eidx = (i_idx * _N + j_idx).reshape(_A // _ROWS_PER_DMA, _ROWS_PER_DMA)
    he_flat = h_edges.reshape(_B, _N * _N, _E)

    s_all, stop_tab = _node_tables(h_nodes, w_cat, bias)
    edit_logits = _sc_edit_logits(he_flat, s_all.reshape(_B, _N * 16), w_e_t,
                                  eidx, i_idx, j_idx, edit_b, feas)

    stop = stop_tab[:, 0, 0]
    stop = jnp.where(stop_feas.astype(bool), stop,
                     jnp.full_like(stop, -jnp.inf))
    return jnp.concatenate([edit_logits, stop[:, None]], axis=1)


# speedup vs baseline: 4.0997x; 1.3185x over previous
import functools

import jax
import jax.numpy as jnp
from jax import lax
from jax.experimental import pallas as pl
from jax.experimental.pallas import tpu as pltpu
from jax.experimental.pallas import tpu_sc as plsc

_B, _N, _D, _E, _A = 32, 128, 256, 16, 4096
_L = 16
_NCHUNK = _A // _L


def _tc_body(he_ref, h_ref, wcat_ref, bias_ref, we_ref, t4_ref, s8t_ref,
             stop_ref):
    h = h_ref[0]
    s8 = jnp.dot(h, wcat_ref[...], preferred_element_type=jnp.float32)
    s8 = s8 + bias_ref[...]
    s8t = s8.T
    s8t_ref[0] = s8t
    stop_ref[...] = jnp.full((1, 1, 128), jnp.sum(s8[:, 8:9]), jnp.float32)
    for c in range(4):
        acc = we_ref[0, c] * he_ref[0, :, 0, :]
        for e in range(1, _E):
            acc = acc + we_ref[e, c] * he_ref[0, :, e, :]
        acc = acc + s8t[4 + c:5 + c, :]
        t4_ref[0, c] = acc


def _tc_tables(he_t, h_nodes, w_cat, bias, w_e):
    return pl.pallas_call(
        _tc_body,
        grid=(_B,),
        in_specs=[
            pl.BlockSpec((1, _N, _E, _N), lambda b: (b, 0, 0, 0)),
            pl.BlockSpec((1, _N, _D), lambda b: (b, 0, 0)),
            pl.BlockSpec((_D, 16), lambda b: (0, 0)),
            pl.BlockSpec((1, 16), lambda b: (0, 0)),
            pl.BlockSpec((_E, 4), lambda b: (0, 0)),
        ],
        out_specs=[
            pl.BlockSpec((1, 4, _N, _N), lambda b: (b, 0, 0, 0)),
            pl.BlockSpec((1, 16, _N), lambda b: (b, 0, 0)),
            pl.BlockSpec((1, 1, 128), lambda b: (b, 0, 0)),
        ],
        out_shape=[
            jax.ShapeDtypeStruct((_B, 4, _N, _N), jnp.float32),
            jax.ShapeDtypeStruct((_B, 16, _N), jnp.float32),
            jax.ShapeDtypeStruct((_B, 1, 128), jnp.float32),
        ],
    )(he_t, h_nodes, w_cat, bias, w_e)


def _sc_edit_logits(t4, s8t, i_idx, j_idx, c_idx, feas):
    info = plsc.get_sparse_core_info()
    mesh = plsc.VectorSubcoreMesh(core_axis_name="c", subcore_axis_name="s")

    @functools.partial(
        pl.kernel,
        out_type=jax.ShapeDtypeStruct((_B, _A // 128, 128), jnp.float32),
        mesh=mesh,
        compiler_params=pltpu.CompilerParams(
            needs_layout_passes=False, use_tc_tiling_on_sc=False),
        scratch_types=[
            pltpu.VMEM((_A,), jnp.int32),
            pltpu.VMEM((_A,), jnp.int32),
            pltpu.VMEM((_A,), jnp.int32),
            pltpu.VMEM((_A,), jnp.int32),
            pltpu.VMEM((4, _N, _N), jnp.float32),
            pltpu.VMEM((16, _N), jnp.float32),
            pltpu.VMEM((_A // 128, 128), jnp.float32),
        ],
    )
    def k(t4_hbm, s8t_hbm, i_hbm, j_hbm, c_hbm, feas_hbm,
          out_hbm, i_v, j_v, c_v, feas_v, t4_v, s8t_v, out_v):
        wid = lax.axis_index("s") * info.num_cores + lax.axis_index("c")
        pltpu.sync_copy(i_hbm, i_v)
        pltpu.sync_copy(j_hbm, j_v)
        pltpu.sync_copy(c_hbm, c_v)
        pltpu.sync_copy(feas_hbm.at[wid], feas_v)
        pltpu.sync_copy(t4_hbm.at[wid], t4_v)
        pltpu.sync_copy(s8t_hbm.at[wid], s8t_v)

        minf = jnp.float32(-jnp.inf)

        def chunk(q, _):
            a0 = q * _L
            iv = i_v[pl.ds(a0, _L)]
            jv = j_v[pl.ds(a0, _L)]
            cv = c_v[pl.ds(a0, _L)]
            acc = plsc.load_gather(t4_v, [cv, iv, jv])
            acc = acc + plsc.load_gather(s8t_v, [cv, iv])
            fv = feas_v[pl.ds(a0, _L)]
            res = jnp.where(fv != 0, acc, minf)
            out_v[q // 8, pl.ds((q % 8) * _L, _L)] = res
            return 0

        lax.fori_loop(0, _NCHUNK, chunk, 0)
        pltpu.sync_copy(out_v, out_hbm.at[wid])

    return k(t4, s8t, i_idx, j_idx, c_idx, feas)


def kernel(h_nodes, h_edges, W_edit, b_edit, W_stop, b_stop, edit_ij, edit_b,
           feas, stop_feas):
    w_cat = jnp.concatenate(
        [W_edit[:_D], W_edit[_D:2 * _D], W_stop / _N,
         jnp.zeros((_D, 7), jnp.float32)], axis=1)
    bias = jnp.concatenate(
        [b_edit, jnp.zeros((4,), jnp.float32), b_stop / _N,
         jnp.zeros((7,), jnp.float32)])[None, :]
    w_e = W_edit[2 * _D:]

    he_t = jnp.transpose(h_edges, (0, 1, 3, 2))

    t4, s8t, stop_tab = _tc_tables(he_t, h_nodes, w_cat, bias, w_e)
    edit_logits = _sc_edit_logits(t4, s8t, edit_ij[0], edit_ij[1], edit_b,
                                  feas)

    stop = stop_tab[:, 0, 0]
    stop = jnp.where(stop_feas.astype(bool), stop,
                     jnp.full_like(stop, -jnp.inf))
    return jnp.concatenate(
        [edit_logits.reshape(_B, _A), stop[:, None]], axis=1)


# speedup vs baseline: 11.8226x; 2.8838x over previous
import functools

import jax
import jax.numpy as jnp
from jax import lax
from jax.experimental import pallas as pl
from jax.experimental.pallas import tpu as pltpu
from jax.experimental.pallas import tpu_sc as plsc

_B, _N, _D, _E, _A = 32, 128, 256, 16, 4096
_L = 16
_NCHUNK = _A // _L


def _tc_body(he_ref, h_ref, wcat_ref, bias_ref, w4t_ref, t4_ref, s8t_ref,
             stop_ref):
    h = h_ref[0]
    s8 = jnp.dot(h, wcat_ref[...], preferred_element_type=jnp.float32)
    s8 = s8 + bias_ref[...]
    s8t = s8.T
    s8t_ref[0] = s8t
    stop_ref[...] = jnp.full((1, 1, 128), jnp.sum(s8[:, 8:9]), jnp.float32)
    sj = jnp.broadcast_to(s8t[None, 4:8, :], (8, 4, _N)).reshape(32, _N)
    w4t = w4t_ref[...]
    for g in range(_N // 8):
        m = he_ref[0, pl.ds(8 * g, 8)].reshape(128, _N)
        out = jnp.dot(w4t, m, preferred_element_type=jnp.float32) + sj
        t4_ref[0, pl.ds(8 * g, 8)] = out.reshape(8, 4, _N)


def _tc_tables(he_t, h_nodes, w_cat, bias, w_e):
    return pl.pallas_call(
        _tc_body,
        grid=(_B,),
        in_specs=[
            pl.BlockSpec((1, _N, _E, _N), lambda b: (b, 0, 0, 0)),
            pl.BlockSpec((1, _N, _D), lambda b: (b, 0, 0)),
            pl.BlockSpec((_D, 16), lambda b: (0, 0)),
            pl.BlockSpec((1, 16), lambda b: (0, 0)),
            pl.BlockSpec((32, 128), lambda b: (0, 0)),
        ],
        out_specs=[
            pl.BlockSpec((1, _N, 4, _N), lambda b: (b, 0, 0, 0)),
            pl.BlockSpec((1, 16, _N), lambda b: (b, 0, 0)),
            pl.BlockSpec((1, 1, 128), lambda b: (b, 0, 0)),
        ],
        out_shape=[
            jax.ShapeDtypeStruct((_B, _N, 4, _N), jnp.float32),
            jax.ShapeDtypeStruct((_B, 16, _N), jnp.float32),
            jax.ShapeDtypeStruct((_B, 1, 128), jnp.float32),
        ],
    )(he_t, h_nodes, w_cat, bias, w_e)


def _sc_edit_logits(t4, s8t, i_idx, j_idx, c_idx, feas):
    info = plsc.get_sparse_core_info()
    mesh = plsc.VectorSubcoreMesh(core_axis_name="c", subcore_axis_name="s")

    @functools.partial(
        pl.kernel,
        out_type=jax.ShapeDtypeStruct((_B, _A // 128, 128), jnp.float32),
        mesh=mesh,
        compiler_params=pltpu.CompilerParams(
            needs_layout_passes=False, use_tc_tiling_on_sc=False),
        scratch_types=[
            pltpu.VMEM((_A,), jnp.int32),
            pltpu.VMEM((_A,), jnp.int32),
            pltpu.VMEM((_A,), jnp.int32),
            pltpu.VMEM((_A,), jnp.int32),
            pltpu.VMEM((_N, 4, _N), jnp.float32),
            pltpu.VMEM((16, _N), jnp.float32),
            pltpu.VMEM((_A // 128, 128), jnp.float32),
        ],
    )
    def k(t4_hbm, s8t_hbm, i_hbm, j_hbm, c_hbm, feas_hbm,
          out_hbm, i_v, j_v, c_v, feas_v, t4_v, s8t_v, out_v):
        wid = lax.axis_index("s") * info.num_cores + lax.axis_index("c")
        pltpu.sync_copy(i_hbm, i_v)
        pltpu.sync_copy(j_hbm, j_v)
        pltpu.sync_copy(c_hbm, c_v)
        pltpu.sync_copy(feas_hbm.at[wid], feas_v)
        pltpu.sync_copy(t4_hbm.at[wid], t4_v)
        pltpu.sync_copy(s8t_hbm.at[wid], s8t_v)

        minf = jnp.float32(-jnp.inf)

        def chunk(q, _):
            a0 = q * _L
            iv = i_v[pl.ds(a0, _L)]
            jv = j_v[pl.ds(a0, _L)]
            cv = c_v[pl.ds(a0, _L)]
            acc = plsc.load_gather(t4_v, [iv, cv, jv])
            acc = acc + plsc.load_gather(s8t_v, [cv, iv])
            fv = feas_v[pl.ds(a0, _L)]
            res = jnp.where(fv != 0, acc, minf)
            out_v[q // 8, pl.ds((q % 8) * _L, _L)] = res
            return 0

        lax.fori_loop(0, _NCHUNK, chunk, 0)
        pltpu.sync_copy(out_v, out_hbm.at[wid])

    return k(t4, s8t, i_idx, j_idx, c_idx, feas)


def kernel(h_nodes, h_edges, W_edit, b_edit, W_stop, b_stop, edit_ij, edit_b,
           feas, stop_feas):
    w_cat = jnp.concatenate(
        [W_edit[:_D], W_edit[_D:2 * _D], W_stop / _N,
         jnp.zeros((_D, 7), jnp.float32)], axis=1)
    bias = jnp.concatenate(
        [b_edit, jnp.zeros((4,), jnp.float32), b_stop / _N,
         jnp.zeros((7,), jnp.float32)])[None, :]
    w_e = W_edit[2 * _D:]
    w4t = jnp.kron(jnp.eye(8, dtype=jnp.float32), w_e.T)

    he_t = jnp.transpose(h_edges, (0, 1, 3, 2))

    t4, s8t, stop_tab = _tc_tables(he_t, h_nodes, w_cat, bias, w4t)
    edit_logits = _sc_edit_logits(t4, s8t, edit_ij[0], edit_ij[1], edit_b,
                                  feas)

    stop = stop_tab[:, 0, 0]
    stop = jnp.where(stop_feas.astype(bool), stop,
                     jnp.full_like(stop, -jnp.inf))
    return jnp.concatenate(
        [edit_logits.reshape(_B, _A), stop[:, None]], axis=1)
